# baseline (speedup 1.0000x reference)
import functools

import jax
import jax.numpy as jnp
from jax import lax
from jax.experimental import pallas as pl
from jax.experimental.pallas import tpu as pltpu
from jax.experimental.pallas import tpu_sc as plsc

_TEMP = 1.0
_TOP_K = 50
_TOP_P = 0.95
_VB = 2048
_R = 8


def _proj_kernel(a1_ref, a2_ref, wr_ref, wi_ref, amp_ref):
    dn = (((1,), (1,)), ((), ()))
    d1 = lax.dot_general(a1_ref[:], wr_ref[:], dn,
                         preferred_element_type=jnp.float32)
    d2 = lax.dot_general(a2_ref[:], wi_ref[:], dn,
                         preferred_element_type=jnp.float32)
    amp = d1 + d2
    m = amp.shape[0] // 2
    amp_ref[:] = amp[:m] ** 2 + amp[m:] ** 2


def _sc_select_kernel(amp_hbm, out_hbm, x_v, m_v, m2_v, sgid_v, mv_v, midx_v,
                      ggid_v, cand_v, ce_v, out_v, *, k, p):
    nc = 2
    wid = lax.axis_index("s") * nc + lax.axis_index("c")
    pltpu.sync_copy(amp_hbm.at[wid], x_v)

    iota = lax.iota(jnp.int32, 16)
    zi = jnp.zeros((16,), jnp.int32)

    def fold1(j, sacc):
        base = j * 16
        f = x_v[pl.ds(base, 16)]
        s = f
        for kk in range(1, 10):
            w = x_v[pl.ds(base + kk * 10000, 16)]
            f = jnp.maximum(f, w)
            s = s + w
        m_v[pl.ds(base, 16)] = f
        return sacc + s

    sacc = lax.fori_loop(0, 625, fold1, jnp.zeros((16,), jnp.float32))
    s1 = jnp.sum(sacc)

    def fold2(j, macc):
        base = j * 16
        f = m_v[pl.ds(base, 16)]
        for a in range(1, 5):
            f = jnp.maximum(f, m_v[pl.ds(base + a * 2000, 16)])
        m2_v[pl.ds(base, 16)] = f
        return jnp.maximum(macc, f)

    macc = lax.fori_loop(0, 125, fold2, jnp.zeros((16,), jnp.float32))
    vmax = jnp.max(macc)
    vmax_v = jnp.full((16,), vmax)
    hi_init = plsc.bitcast(vmax_v, jnp.int32) + 1

    def count_ge(ref, nvr, thr):
        def cb(j, cacc):
            vals = ref[pl.ds(j * 16, 16)]
            return cacc + plsc.all_reduce_population_count(vals >= thr)
        return lax.fori_loop(0, nvr, cb, zi)

    def bs_count(ref, nvr, lo0, hi0):
        def body(_, carry):
            lo, hi = carry
            mid = lo + lax.shift_right_logical(hi - lo, 1)
            cnt = count_ge(ref, nvr, plsc.bitcast(mid, jnp.float32))
            ok = cnt >= k
            return jnp.where(ok, mid, lo), jnp.where(ok, hi, mid)
        return lax.fori_loop(0, 31, body, (lo0, hi0))[0]

    s50m2 = bs_count(m2_v, 125, zi, hi_init)
    s50m2_f = plsc.bitcast(s50m2, jnp.float32)

    def comp_s(j, widx):
        vals = m2_v[pl.ds(j * 16, 16)]
        mask = vals >= s50m2_f
        pos = plsc.cumsum(jnp.where(mask, 1, 0)) - 1 + widx
        plsc.store_scatter(sgid_v, [pos], iota + j * 16, mask=mask)
        return widx + plsc.all_reduce_population_count(mask)

    ws = lax.fori_loop(0, 125, comp_s, zi)

    for t in range(4):
        sv = sgid_v[pl.ds(t * 16, 16)]
        validt = (iota + t * 16) < ws
        sv = jnp.minimum(jnp.maximum(jnp.where(validt, sv, 0), 0), 1999)
        for a in range(5):
            midx = sv + 2000 * a
            mv = plsc.load_gather(m_v, [midx])
            mv_v[pl.ds((t * 5 + a) * 16, 16)] = jnp.where(validt, mv, 0.0)
            midx_v[pl.ds((t * 5 + a) * 16, 16)] = jnp.where(validt, midx, 0)
    for t in range(44):
        mv_v[pl.ds((20 + t) * 16, 16)] = jnp.zeros((16,), jnp.float32)
        midx_v[pl.ds((20 + t) * 16, 16)] = zi

    s50m = bs_count(mv_v, 64, s50m2, hi_init)
    s50m_f = plsc.bitcast(s50m, jnp.float32)

    def comp_g(j, widx):
        vals = mv_v[pl.ds(j * 16, 16)]
        mask = vals >= s50m_f
        pos = plsc.cumsum(jnp.where(mask, 1, 0)) - 1 + widx
        plsc.store_scatter(ggid_v, [pos], midx_v[pl.ds(j * 16, 16)], mask=mask)
        return widx + plsc.all_reduce_population_count(mask)

    wg = lax.fori_loop(0, 64, comp_g, zi)

    for t in range(4):
        gv = ggid_v[pl.ds(t * 16, 16)]
        validt = (iota + t * 16) < wg
        gv = jnp.minimum(jnp.maximum(jnp.where(validt, gv, 0), 0), 9999)
        for kk in range(10):
            eidx = gv + 10000 * kk
            ev = plsc.load_gather(x_v, [eidx])
            cand_v[pl.ds((t * 10 + kk) * 16, 16)] = jnp.where(validt, ev, 0.0)
    for t in range(24):
        cand_v[pl.ds((40 + t) * 16, 16)] = jnp.zeros((16,), jnp.float32)

    v50 = bs_count(cand_v, 64, s50m, hi_init)
    v50_f = plsc.bitcast(v50, jnp.float32)

    floor = s1 * (1e-06 / x_v.shape[0]) + 1e-30
    floor_v = jnp.full((16,), floor)
    for gi in range(64):
        c = cand_v[pl.ds(gi * 16, 16)]
        e = jnp.where(c >= v50_f, c + floor_v, 0.0)
        ce_v[pl.ds(gi * 16, 16)] = e

    def sum_e(j, acc):
        return acc + ce_v[pl.ds(j * 16, 16)]

    z = jnp.sum(lax.fori_loop(0, 64, sum_e, jnp.zeros((16,), jnp.float32)))
    pz = p * z

    def bs2_body(_, carry):
        lo, hi = carry
        mid = lo + lax.shift_right_logical(hi - lo, 1)
        thr = plsc.bitcast(mid, jnp.float32)

        def mb(j, acc):
            c = cand_v[pl.ds(j * 16, 16)]
            e = ce_v[pl.ds(j * 16, 16)]
            return acc + jnp.where(c > thr, e, 0.0)

        gs = jnp.sum(lax.fori_loop(0, 64, mb, jnp.zeros((16,), jnp.float32)))
        ok = jnp.full((16,), gs >= pz)
        return jnp.where(ok, mid, lo), jnp.where(ok, hi, mid)

    cut = lax.fori_loop(0, 31, bs2_body, (v50 - 1, hi_init - 1))[0]

    out_v[...] = jnp.where(iota == 0, v50, jnp.where(iota == 1, cut, 0))
    pltpu.sync_copy(out_v, out_hbm.at[wid])


def _sel_thresh_kernel(amp_ref, bias_ref, g_ref, stats_ref, logits_ref,
                       lp_ref, probs_ref, tok_ref, *, temp):
    x = amp_ref[:]
    r, v = x.shape
    s1 = jnp.sum(x, axis=-1, keepdims=True)
    vmax = jnp.max(x, axis=-1, keepdims=True)
    floor = (s1 / v) * 1e-06 + 1e-30
    xi = lax.bitcast_convert_type(x, jnp.int32)
    v50b = stats_ref[:, 0:1]
    cutb = stats_ref[:, 1:2]
    keep1 = xi >= v50b
    keep2 = keep1 & (xi > cutb)

    logits = jnp.log(x + floor) + bias_ref[:]
    if temp != 1.0:
        logits = logits / max(temp, 1e-08)
    logits_ref[:] = logits
    lp_ref[:] = logits - jnp.log(s1 + v * floor)

    e2 = jnp.where(keep2, (x + floor) / (vmax + floor), 0.0)
    zk = jnp.sum(e2, axis=-1, keepdims=True)
    probs_ref[:] = e2 / zk

    y = jnp.where(keep2, logits + g_ref[:], -jnp.inf)
    ymax = jnp.max(y, axis=-1, keepdims=True)
    iota = lax.broadcasted_iota(jnp.int32, (r, v), 1)
    tok_ref[:] = jnp.min(jnp.where(y == ymax, iota, v), axis=-1, keepdims=True)


def _sel_kernel(amp_ref, bias_ref, g_ref, logits_ref, lp_ref, probs_ref,
                tok_ref, *, k, p, temp):
    x = amp_ref[:]
    r, v = x.shape
    s1 = jnp.sum(x, axis=-1, keepdims=True)
    vmax = jnp.max(x, axis=-1, keepdims=True)
    floor = (s1 / v) * 1e-06 + 1e-30
    xi = lax.bitcast_convert_type(x, jnp.int32)

    def bs1(_, carry):
        lo, hi = carry
        mid = lo + (hi - lo) // 2
        cnt = jnp.sum(jnp.where(xi >= mid, 1.0, 0.0), axis=-1, keepdims=True)
        ok = cnt >= k
        return jnp.where(ok, mid, lo), jnp.where(ok, hi, mid)

    lo0 = jnp.zeros((r, 1), jnp.int32)
    hi0 = lax.bitcast_convert_type(vmax, jnp.int32) + 1
    v50b, _ = lax.fori_loop(0, 31, bs1, (lo0, hi0))
    keep1 = xi >= v50b

    logits = jnp.log(x + floor) + bias_ref[:]
    if temp != 1.0:
        logits = logits / max(temp, 1e-08)
    logits_ref[:] = logits
    lp_ref[:] = logits - jnp.log(s1 + v * floor)

    e = jnp.where(keep1, (x + floor) / (vmax + floor), 0.0)
    z = jnp.sum(e, axis=-1, keepdims=True)
    pz = p * z

    def bs2_cond(carry):
        lo, hi = carry
        return jnp.any(hi - lo > 1)

    def bs2_body(carry):
        lo, hi = carry
        mid = lo + (hi - lo) // 2
        gsum = jnp.sum(jnp.where(xi > mid, e, 0.0), axis=-1, keepdims=True)
        ok = gsum >= pz
        return jnp.where(ok, mid, lo), jnp.where(ok, hi, mid)

    cutb, _ = lax.while_loop(
        bs2_cond, bs2_body,
        (v50b - 1, lax.bitcast_convert_type(vmax, jnp.int32)))
    keep2 = keep1 & (xi > cutb)

    e2 = jnp.where(keep2, e, 0.0)
    zk = jnp.sum(e2, axis=-1, keepdims=True)
    probs_ref[:] = e2 / zk

    y = jnp.where(keep2, logits + g_ref[:], -jnp.inf)
    ymax = jnp.max(y, axis=-1, keepdims=True)
    iota = lax.broadcasted_iota(jnp.int32, (r, v), 1)
    tok_ref[:] = jnp.min(jnp.where(y == ymax, iota, v), axis=-1, keepdims=True)


def kernel(psi_real, psi_imag, W_real, W_imag, bias):
    b, s, d = psi_real.shape
    v = W_real.shape[0]
    m = b * s
    pr = psi_real.reshape(m, d)
    pi = psi_imag.reshape(m, d)
    a1 = jnp.concatenate([pr, pi], axis=0)
    a2 = jnp.concatenate([-pi, pr], axis=0)
    g = jax.random.gumbel(jax.random.key(42), (b, s, v), jnp.float32)
    g = g.reshape(m, v)

    nb = pl.cdiv(v, _VB)
    amp_sq = pl.pallas_call(
        _proj_kernel,
        grid=(nb,),
        in_specs=[
            pl.BlockSpec((2 * m, d), lambda i: (0, 0)),
            pl.BlockSpec((2 * m, d), lambda i: (0, 0)),
            pl.BlockSpec((_VB, d), lambda i: (i, 0)),
            pl.BlockSpec((_VB, d), lambda i: (i, 0)),
        ],
        out_specs=pl.BlockSpec((m, _VB), lambda i: (0, i)),
        out_shape=jax.ShapeDtypeStruct((m, v), jnp.float32),
        compiler_params=pltpu.CompilerParams(
            dimension_semantics=("arbitrary",)),
    )(a1, a2, W_real, W_imag)

    use_sc = (m == 32 and v == 100000 and _TOP_K == 50 and _TEMP == 1.0)
    if use_sc:
        mesh = plsc.VectorSubcoreMesh(
            core_axis_name="c", subcore_axis_name="s", num_cores=2)
        stats = pl.kernel(
            functools.partial(_sc_select_kernel, k=_TOP_K, p=_TOP_P),
            mesh=mesh,
            out_type=jax.ShapeDtypeStruct((m, 16), jnp.int32),
            compiler_params=pltpu.CompilerParams(needs_layout_passes=False),
            scratch_types=[
                pltpu.VMEM((100000,), jnp.float32),
                pltpu.VMEM((10000,), jnp.float32),
                pltpu.VMEM((2000,), jnp.float32),
                pltpu.VMEM((64,), jnp.int32),
                pltpu.VMEM((1024,), jnp.float32),
                pltpu.VMEM((1024,), jnp.int32),
                pltpu.VMEM((64,), jnp.int32),
                pltpu.VMEM((1024,), jnp.float32),
                pltpu.VMEM((1024,), jnp.float32),
                pltpu.VMEM((16,), jnp.int32),
            ],
        )(amp_sq)
        logits, log_probs, probs, tokens = pl.pallas_call(
            functools.partial(_sel_thresh_kernel, temp=_TEMP),
            grid=(m // _R,),
            in_specs=[
                pl.BlockSpec((_R, v), lambda i: (i, 0)),
                pl.BlockSpec((1, v), lambda i: (0, 0)),
                pl.BlockSpec((_R, v), lambda i: (i, 0)),
                pl.BlockSpec((_R, 16), lambda i: (i, 0)),
            ],
            out_specs=[
                pl.BlockSpec((_R, v), lambda i: (i, 0)),
                pl.BlockSpec((_R, v), lambda i: (i, 0)),
                pl.BlockSpec((_R, v), lambda i: (i, 0)),
                pl.BlockSpec((_R, 1), lambda i: (i, 0)),
            ],
            out_shape=[
                jax.ShapeDtypeStruct((m, v), jnp.float32),
                jax.ShapeDtypeStruct((m, v), jnp.float32),
                jax.ShapeDtypeStruct((m, v), jnp.float32),
                jax.ShapeDtypeStruct((m, 1), jnp.int32),
            ],
            compiler_params=pltpu.CompilerParams(
                dimension_semantics=("arbitrary",)),
        )(amp_sq, bias.reshape(1, v), g, stats)
        shape3 = (b, s, v)
        return (logits.reshape(shape3), log_probs.reshape(shape3),
                amp_sq.reshape(shape3), tokens.reshape(b, s),
                probs.reshape(shape3))

    sel = functools.partial(_sel_kernel, k=_TOP_K, p=_TOP_P, temp=_TEMP)
    logits, log_probs, probs, tokens = pl.pallas_call(
        sel,
        grid=(m // _R,),
        in_specs=[
            pl.BlockSpec((_R, v), lambda i: (i, 0)),
            pl.BlockSpec((1, v), lambda i: (0, 0)),
            pl.BlockSpec((_R, v), lambda i: (i, 0)),
        ],
        out_specs=[
            pl.BlockSpec((_R, v), lambda i: (i, 0)),
            pl.BlockSpec((_R, v), lambda i: (i, 0)),
            pl.BlockSpec((_R, v), lambda i: (i, 0)),
            pl.BlockSpec((_R, 1), lambda i: (i, 0)),
        ],
        out_shape=[
            jax.ShapeDtypeStruct((m, v), jnp.float32),
            jax.ShapeDtypeStruct((m, v), jnp.float32),
            jax.ShapeDtypeStruct((m, v), jnp.float32),
            jax.ShapeDtypeStruct((m, 1), jnp.int32),
        ],
        compiler_params=pltpu.CompilerParams(
            dimension_semantics=("arbitrary",)),
    )(amp_sq, bias.reshape(1, v), g)

    shape3 = (b, s, v)
    return (logits.reshape(shape3), log_probs.reshape(shape3),
            amp_sq.reshape(shape3), tokens.reshape(b, s),
            probs.reshape(shape3))
